# Initial kernel scaffold; baseline (speedup 1.0000x reference)
import jax
import jax.numpy as jnp
from jax.experimental import pallas as pl


def kernel(x, in_proj_w, in_proj_b, out_proj_w, out_proj_b, ln1_g, ln1_b, ln2_g, ln2_b, gate_w, gate_b, expert_w, expert_b):
    raise NotImplementedError("write your pallas kernel here")


# speedup vs baseline: 2.3971x; 2.3971x over previous
import functools

import jax
import jax.numpy as jnp
from jax.experimental import pallas as pl
from jax.experimental.pallas import tpu as pltpu

B, S, D, H, E, K = 2, 2048, 1024, 16, 8, 2
DH = D // H
N = B * S
EPS = 1e-5


HPG = 2
HD2 = HPG * DH
QC = 512


def _attn_kernel(x_ref, wq_ref, wk_ref, wv_ref, bq_ref, o_ref):
    x_bf = x_ref[0].astype(jnp.bfloat16)
    wq = wq_ref[0].astype(jnp.bfloat16)
    wk = wk_ref[0].astype(jnp.bfloat16)
    wv = wv_ref[0].astype(jnp.bfloat16)
    q2 = jnp.dot(x_bf, wq, preferred_element_type=jnp.float32)
    q2 = (q2 + bq_ref[0]) * (1.0 / jnp.sqrt(jnp.float32(DH)))
    k2 = jnp.dot(x_bf, wk, preferred_element_type=jnp.float32)
    v2 = jnp.dot(x_bf, wv, preferred_element_type=jnp.float32)
    q2 = q2.astype(jnp.bfloat16)
    k2 = k2.astype(jnp.bfloat16)
    v2 = v2.astype(jnp.bfloat16)
    for hh in range(HPG):
        k_h = k2[:, hh * DH:(hh + 1) * DH]
        v_h = v2[:, hh * DH:(hh + 1) * DH]
        for c in range(S // QC):
            q_h = q2[c * QC:(c + 1) * QC, hh * DH:(hh + 1) * DH]
            scores = jax.lax.dot_general(
                q_h, k_h, (((1,), (1,)), ((), ())),
                preferred_element_type=jnp.float32)
            m = jnp.max(scores, axis=1, keepdims=True)
            p = jnp.exp(scores - m)
            attn = (p / jnp.sum(p, axis=1, keepdims=True)).astype(jnp.bfloat16)
            o = jnp.dot(attn, v_h, preferred_element_type=jnp.float32)
            o_ref[0, c * QC:(c + 1) * QC, hh * DH:(hh + 1) * DH] = o


def _run_attention(x, wq_r, wk_r, wv_r, bq):
    return pl.pallas_call(
        _attn_kernel,
        grid=(B, H // HPG),
        in_specs=[
            pl.BlockSpec((1, S, D), lambda b, g: (b, 0, 0)),
            pl.BlockSpec((1, D, HD2), lambda b, g: (g, 0, 0)),
            pl.BlockSpec((1, D, HD2), lambda b, g: (g, 0, 0)),
            pl.BlockSpec((1, D, HD2), lambda b, g: (g, 0, 0)),
            pl.BlockSpec((1, 1, HD2), lambda b, g: (g, 0, 0)),
        ],
        out_specs=pl.BlockSpec((1, S, HD2), lambda b, g: (b, 0, g)),
        out_shape=jax.ShapeDtypeStruct((B, S, D), jnp.float32),
        compiler_params=pltpu.CompilerParams(
            dimension_semantics=("arbitrary", "arbitrary")),
    )(x, wq_r, wk_r, wv_r, bq)


TB2 = 512


def _mid_kernel(o_ref, x_ref, wo_ref, beff_ref, g1_ref, b1_ref,
                gw_ref, gb_ref, h_ref, gate_ref, wfull_ref):
    o_bf = o_ref[...].astype(jnp.bfloat16)
    wo = wo_ref[...].astype(jnp.bfloat16)
    ao = jnp.dot(o_bf, wo, preferred_element_type=jnp.float32) + beff_ref[0]
    r = x_ref[...] + ao
    mu = jnp.mean(r, axis=1, keepdims=True)
    c = r - mu
    var = jnp.mean(c * c, axis=1, keepdims=True)
    h = c / jnp.sqrt(var + EPS) * g1_ref[0] + b1_ref[0]
    h_ref[...] = h
    logits = jax.lax.dot_general(
        h.astype(jnp.bfloat16), gw_ref[...].astype(jnp.bfloat16),
        (((1,), (0,)), ((), ())),
        preferred_element_type=jnp.float32) + gb_ref[0]
    lm = jnp.max(logits, axis=1, keepdims=True)
    pe = jnp.exp(logits - lm)
    gate = pe / jnp.sum(pe, axis=1, keepdims=True)
    gate_ref[...] = gate
    iot = jax.lax.broadcasted_iota(jnp.int32, (TB2, E), 1)
    i1 = jnp.argmax(gate, axis=1)
    m1 = jnp.max(gate, axis=1)
    mask1 = iot == i1[:, None]
    g2 = jnp.where(mask1, -1.0, gate)
    i2 = jnp.argmax(g2, axis=1)
    m2 = jnp.max(g2, axis=1)
    ws = m1 + m2
    w1 = (m1 / ws)[:, None]
    w2 = (m2 / ws)[:, None]
    wfull_ref[...] = jnp.where(mask1, w1, 0.0) + jnp.where(
        iot == i2[:, None], w2, 0.0)


def _run_mid(o2, x2, wo_t, b_eff, ln1_g, ln1_b, gate_w, gate_b):
    return pl.pallas_call(
        _mid_kernel,
        grid=(N // TB2,),
        in_specs=[
            pl.BlockSpec((TB2, D), lambda i: (i, 0)),
            pl.BlockSpec((TB2, D), lambda i: (i, 0)),
            pl.BlockSpec((D, D), lambda i: (0, 0)),
            pl.BlockSpec((1, D), lambda i: (0, 0)),
            pl.BlockSpec((1, D), lambda i: (0, 0)),
            pl.BlockSpec((1, D), lambda i: (0, 0)),
            pl.BlockSpec((D, E), lambda i: (0, 0)),
            pl.BlockSpec((1, E), lambda i: (0, 0)),
        ],
        out_specs=[
            pl.BlockSpec((TB2, D), lambda i: (i, 0)),
            pl.BlockSpec((TB2, E), lambda i: (i, 0)),
            pl.BlockSpec((TB2, E), lambda i: (i, 0)),
        ],
        out_shape=[
            jax.ShapeDtypeStruct((N, D), jnp.float32),
            jax.ShapeDtypeStruct((N, E), jnp.float32),
            jax.ShapeDtypeStruct((N, E), jnp.float32),
        ],
        compiler_params=pltpu.CompilerParams(
            dimension_semantics=("arbitrary",)),
    )(o2, x2, wo_t, b_eff, ln1_g, ln1_b, gate_w, gate_b)


TB3 = 2048


def _moe_kernel(h_ref, ew_ref, eb_ref, wf_ref, g2_ref, b2_ref, out_ref):
    e = pl.program_id(1)

    @pl.when(e == 0)
    def _():
        out_ref[...] = jnp.zeros_like(out_ref)

    h_bf = h_ref[...].astype(jnp.bfloat16)
    ew = ew_ref[0].astype(jnp.bfloat16)
    y = jnp.dot(h_bf, ew, preferred_element_type=jnp.float32) + eb_ref[0]
    sel = (jax.lax.broadcasted_iota(jnp.int32, (TB3, E), 1) == e)
    w_col = jnp.sum(jnp.where(sel, wf_ref[...], 0.0), axis=1)
    out_ref[...] += w_col[:, None] * y

    @pl.when(e == E - 1)
    def _():
        r = h_ref[...] + out_ref[...]
        mu = jnp.mean(r, axis=1, keepdims=True)
        c = r - mu
        var = jnp.mean(c * c, axis=1, keepdims=True)
        out_ref[...] = c / jnp.sqrt(var + EPS) * g2_ref[0] + b2_ref[0]


def _run_moe(h, expert_w, expert_b3, wfull, ln2_g, ln2_b):
    return pl.pallas_call(
        _moe_kernel,
        grid=(N // TB3, E),
        in_specs=[
            pl.BlockSpec((TB3, D), lambda t, e: (t, 0)),
            pl.BlockSpec((1, D, D), lambda t, e: (e, 0, 0)),
            pl.BlockSpec((1, 1, D), lambda t, e: (e, 0, 0)),
            pl.BlockSpec((TB3, E), lambda t, e: (t, 0)),
            pl.BlockSpec((1, D), lambda t, e: (0, 0)),
            pl.BlockSpec((1, D), lambda t, e: (0, 0)),
        ],
        out_specs=pl.BlockSpec((TB3, D), lambda t, e: (t, 0)),
        out_shape=jax.ShapeDtypeStruct((N, D), jnp.float32),
        compiler_params=pltpu.CompilerParams(
            dimension_semantics=("arbitrary", "arbitrary")),
    )(h, expert_w, expert_b3, wfull, ln2_g, ln2_b)


@jax.jit
def kernel(x, in_proj_w, in_proj_b, out_proj_w, out_proj_b, ln1_g, ln1_b,
           ln2_g, ln2_b, gate_w, gate_b, expert_w, expert_b):
    wq_r = in_proj_w[:D].T.reshape(D, H // HPG, HD2).transpose(1, 0, 2)
    wk_r = in_proj_w[D:2 * D].T.reshape(D, H // HPG, HD2).transpose(1, 0, 2)
    wv_r = in_proj_w[2 * D:].T.reshape(D, H // HPG, HD2).transpose(1, 0, 2)
    bq = in_proj_b[:D].reshape(H // HPG, 1, HD2)
    bv = in_proj_b[2 * D:]
    b_eff = (out_proj_b + bv @ out_proj_w.T).reshape(1, D)

    o = _run_attention(x, wq_r, wk_r, wv_r, bq)

    o2 = o.reshape(N, D)
    x2 = x.reshape(N, D)
    h, gate, wfull = _run_mid(
        o2, x2, out_proj_w.T, b_eff, ln1_g.reshape(1, D), ln1_b.reshape(1, D),
        gate_w, gate_b.reshape(1, E))

    out = _run_moe(h, expert_w, expert_b.reshape(E, 1, D), wfull,
                   ln2_g.reshape(1, D), ln2_b.reshape(1, D))
    return out.reshape(B, S, D), gate.reshape(B, S, E)
